# Initial kernel scaffold; baseline (speedup 1.0000x reference)
import jax
import jax.numpy as jnp
from jax.experimental import pallas as pl


def kernel(x, edge_index, W1a, b1a, W1b, b1b, gamma, beta, W2, b2):
    raise NotImplementedError("write your pallas kernel here")


# speedup vs baseline: 14.9599x; 14.9599x over previous
import functools

import jax
import jax.numpy as jnp
from jax import lax
from jax.experimental import pallas as pl
from jax.experimental.pallas import tpu as pltpu
from jax.experimental.pallas import tpu_sc as plsc

N_NODES = 10000
D_FEAT = 128
HIDDEN = 16

NC = 2
NS = 16
NW = NC * NS
CHUNK = 128
NCHUNKS = 80
EDGES_PER_TILE = CHUNK * NCHUNKS
E_PAD = EDGES_PER_TILE * NW
NPAD = 10240
ROWS_PER_TILE = NPAD // NS

_MESH = plsc.VectorSubcoreMesh(
    core_axis_name="c", subcore_axis_name="s", num_cores=NC, num_subcores=NS
)


def _segsum_body(y0_hbm, zeros_hbm, src_hbm, dst_hbm, out_hbm,
                 sidx, didx, rows0, rows1, acc, gsem0, gsem1):
    c = lax.axis_index("c")
    s = lax.axis_index("s")
    wid = c * NS + s
    r0 = s * ROWS_PER_TILE

    pltpu.sync_copy(zeros_hbm.at[pl.ds(r0, ROWS_PER_TILE)],
                    acc.at[pl.ds(r0, ROWS_PER_TILE)])
    pltpu.sync_copy(src_hbm.at[wid], sidx)
    pltpu.sync_copy(dst_hbm.at[wid], didx)
    plsc.subcore_barrier()

    pltpu.async_copy(y0_hbm.at[sidx.at[0]], rows0, gsem0)
    pltpu.async_copy(y0_hbm.at[sidx.at[1]], rows1, gsem1)

    def step(i, carry):
        j0 = i * 2
        pltpu.make_async_copy(y0_hbm.at[sidx.at[j0]], rows0, gsem0).wait()
        pltpu.sync_copy(rows0, acc.at[didx.at[j0]], add=True)

        @pl.when(j0 + 2 < NCHUNKS)
        def _():
            pltpu.async_copy(y0_hbm.at[sidx.at[j0 + 2]], rows0, gsem0)

        j1 = j0 + 1
        pltpu.make_async_copy(y0_hbm.at[sidx.at[j1]], rows1, gsem1).wait()
        pltpu.sync_copy(rows1, acc.at[didx.at[j1]], add=True)

        @pl.when(j1 + 2 < NCHUNKS)
        def _():
            pltpu.async_copy(y0_hbm.at[sidx.at[j1 + 2]], rows1, gsem1)

        return carry

    lax.fori_loop(0, NCHUNKS // 2, step, 0)

    plsc.subcore_barrier()
    pltpu.sync_copy(acc.at[pl.ds(r0, ROWS_PER_TILE)],
                    out_hbm.at[c, pl.ds(r0, ROWS_PER_TILE)])


_segsum = pl.kernel(
    _segsum_body,
    out_type=jax.ShapeDtypeStruct((NC, NPAD, HIDDEN), jnp.float32),
    mesh=_MESH,
    scratch_types=[
        pltpu.VMEM((NCHUNKS, CHUNK), jnp.int32),
        pltpu.VMEM((NCHUNKS, CHUNK), jnp.int32),
        pltpu.VMEM((CHUNK, HIDDEN), jnp.float32),
        pltpu.VMEM((CHUNK, HIDDEN), jnp.float32),
        pltpu.VMEM_SHARED((NPAD, HIDDEN), jnp.float32),
        pltpu.SemaphoreType.DMA,
        pltpu.SemaphoreType.DMA,
    ],
    compiler_params=pltpu.CompilerParams(use_tc_tiling_on_sc=False),
)


def _proj_body(x_ref, w_ref, o_ref):
    o_ref[...] = jnp.dot(x_ref[...], w_ref[...],
                         preferred_element_type=jnp.float32)


def _mid_body(y0_ref, agg_ref, b1a_ref, w1b_ref, b1b_ref, gamma_ref,
              beta_ref, w2_ref, o_ref):
    pre = (y0_ref[...] + agg_ref[0, :N_NODES, :] + agg_ref[1, :N_NODES, :]
           + b1a_ref[...])
    a = jnp.maximum(pre, 0.0)
    h = jnp.dot(a, w1b_ref[...], preferred_element_type=jnp.float32) \
        + b1b_ref[...]
    mean = jnp.mean(h, axis=0, keepdims=True)
    var = jnp.mean((h - mean) ** 2, axis=0, keepdims=True)
    hn = (h - mean) * lax.rsqrt(var + 1e-5) * gamma_ref[...] + beta_ref[...]
    hn = jnp.maximum(hn, 0.0)
    o_ref[...] = jnp.dot(hn, w2_ref[...], preferred_element_type=jnp.float32)


def _post_body(z0_ref, agg_ref, b2_ref, o_ref):
    t = (z0_ref[...] + agg_ref[0, :N_NODES, :] + agg_ref[1, :N_NODES, :]
         + b2_ref[...])
    m = jnp.max(t, axis=1, keepdims=True)
    lse = jnp.log(jnp.sum(jnp.exp(t - m), axis=1, keepdims=True))
    o_ref[...] = t - m - lse


def kernel(x, edge_index, W1a, b1a, W1b, b1b, gamma, beta, W2, b2):
    src = edge_index[0].astype(jnp.int32)
    dst = edge_index[1].astype(jnp.int32)
    npad_e = E_PAD - src.shape[0]
    pad_dst = N_NODES + (jnp.arange(npad_e, dtype=jnp.int32)
                         % (NPAD - N_NODES))
    src_p = jnp.concatenate(
        [src, jnp.zeros((npad_e,), jnp.int32)]).reshape(NW, NCHUNKS, CHUNK)
    dst_p = jnp.concatenate([dst, pad_dst]).reshape(NW, NCHUNKS, CHUNK)
    zeros_tab = jnp.zeros((NPAD, HIDDEN), jnp.float32)

    y0 = pl.pallas_call(
        _proj_body,
        out_shape=jax.ShapeDtypeStruct((N_NODES, HIDDEN), jnp.float32),
    )(x, W1a)

    agg1 = _segsum(y0, zeros_tab, src_p, dst_p)

    z0 = pl.pallas_call(
        _mid_body,
        out_shape=jax.ShapeDtypeStruct((N_NODES, HIDDEN), jnp.float32),
    )(y0, agg1, b1a.reshape(1, HIDDEN), W1b, b1b.reshape(1, HIDDEN),
      gamma.reshape(1, HIDDEN), beta.reshape(1, HIDDEN), W2)

    agg2 = _segsum(z0, zeros_tab, src_p, dst_p)

    out = pl.pallas_call(
        _post_body,
        out_shape=jax.ShapeDtypeStruct((N_NODES, HIDDEN), jnp.float32),
    )(z0, agg2, b2.reshape(1, HIDDEN))
    return out


# speedup vs baseline: 15.1741x; 1.0143x over previous
import functools

import jax
import jax.numpy as jnp
from jax import lax
from jax.experimental import pallas as pl
from jax.experimental.pallas import tpu as pltpu
from jax.experimental.pallas import tpu_sc as plsc

N_NODES = 10000
D_FEAT = 128
HIDDEN = 16

NC = 2
NS = 16
NW = NC * NS
CHUNK = 128
NCHUNKS = 80
EDGES_PER_TILE = CHUNK * NCHUNKS
E_PAD = EDGES_PER_TILE * NW
NPAD = 10240
ROWS_PER_TILE = NPAD // NS

_MESH = plsc.VectorSubcoreMesh(
    core_axis_name="c", subcore_axis_name="s", num_cores=NC, num_subcores=NS
)


NBUF = 4
PREFETCH = 2


def _segsum_body(y0_hbm, zeros_hbm, src_hbm, dst_hbm, out_hbm,
                 sidx, didx, rows, acc, gsems, ssems):
    c = lax.axis_index("c")
    s = lax.axis_index("s")
    wid = c * NS + s
    r0 = s * ROWS_PER_TILE

    pltpu.sync_copy(zeros_hbm.at[pl.ds(r0, ROWS_PER_TILE)],
                    acc.at[pl.ds(r0, ROWS_PER_TILE)])
    pltpu.sync_copy(src_hbm.at[wid], sidx)
    pltpu.sync_copy(dst_hbm.at[wid], didx)
    plsc.subcore_barrier()

    for b in range(PREFETCH):
        pltpu.async_copy(y0_hbm.at[sidx.at[b]], rows[b], gsems[b])

    def step(i, carry):
        for b0 in range(NBUF):
            j = i * NBUF + b0
            b = b0
            pltpu.make_async_copy(y0_hbm.at[sidx.at[j]], rows[b],
                                  gsems[b]).wait()
            pltpu.async_copy(rows[b], acc.at[didx.at[j]], ssems[b], add=True)

            bn = (b0 + PREFETCH) % NBUF

            @pl.when(j >= NBUF - PREFETCH)
            def _():
                pltpu.make_async_copy(rows[bn], acc.at[didx.at[0]],
                                      ssems[bn]).wait()

            @pl.when(j + PREFETCH < NCHUNKS)
            def _():
                pltpu.async_copy(y0_hbm.at[sidx.at[j + PREFETCH]], rows[bn],
                                 gsems[bn])
        return carry

    lax.fori_loop(0, NCHUNKS // NBUF, step, 0)

    for k in range(NCHUNKS - (NBUF - PREFETCH), NCHUNKS):
        pltpu.make_async_copy(rows[k % NBUF], acc.at[didx.at[0]],
                              ssems[k % NBUF]).wait()

    plsc.subcore_barrier()
    pltpu.sync_copy(acc.at[pl.ds(r0, ROWS_PER_TILE)],
                    out_hbm.at[c, pl.ds(r0, ROWS_PER_TILE)])


_segsum = pl.kernel(
    _segsum_body,
    out_type=jax.ShapeDtypeStruct((NC, NPAD, HIDDEN), jnp.float32),
    mesh=_MESH,
    scratch_types=[
        pltpu.VMEM((NCHUNKS, CHUNK), jnp.int32),
        pltpu.VMEM((NCHUNKS, CHUNK), jnp.int32),
        tuple(pltpu.VMEM((CHUNK, HIDDEN), jnp.float32)
              for _ in range(NBUF)),
        pltpu.VMEM_SHARED((NPAD, HIDDEN), jnp.float32),
        tuple(pltpu.SemaphoreType.DMA for _ in range(NBUF)),
        tuple(pltpu.SemaphoreType.DMA for _ in range(NBUF)),
    ],
    compiler_params=pltpu.CompilerParams(use_tc_tiling_on_sc=False),
)


def _proj_body(x_ref, w_ref, o_ref):
    o_ref[...] = jnp.dot(x_ref[...], w_ref[...],
                         preferred_element_type=jnp.float32)


def _mid_body(y0_ref, agg_ref, b1a_ref, w1b_ref, b1b_ref, gamma_ref,
              beta_ref, w2_ref, o_ref):
    pre = (y0_ref[...] + agg_ref[0, :N_NODES, :] + agg_ref[1, :N_NODES, :]
           + b1a_ref[...])
    a = jnp.maximum(pre, 0.0)
    h = jnp.dot(a, w1b_ref[...], preferred_element_type=jnp.float32) \
        + b1b_ref[...]
    mean = jnp.mean(h, axis=0, keepdims=True)
    var = jnp.mean((h - mean) ** 2, axis=0, keepdims=True)
    hn = (h - mean) * lax.rsqrt(var + 1e-5) * gamma_ref[...] + beta_ref[...]
    hn = jnp.maximum(hn, 0.0)
    o_ref[...] = jnp.dot(hn, w2_ref[...], preferred_element_type=jnp.float32)


def _post_body(z0_ref, agg_ref, b2_ref, o_ref):
    t = (z0_ref[...] + agg_ref[0, :N_NODES, :] + agg_ref[1, :N_NODES, :]
         + b2_ref[...])
    m = jnp.max(t, axis=1, keepdims=True)
    lse = jnp.log(jnp.sum(jnp.exp(t - m), axis=1, keepdims=True))
    o_ref[...] = t - m - lse


def kernel(x, edge_index, W1a, b1a, W1b, b1b, gamma, beta, W2, b2):
    src = edge_index[0].astype(jnp.int32)
    dst = edge_index[1].astype(jnp.int32)
    npad_e = E_PAD - src.shape[0]
    pad_dst = N_NODES + (jnp.arange(npad_e, dtype=jnp.int32)
                         % (NPAD - N_NODES))
    src_p = jnp.concatenate(
        [src, jnp.zeros((npad_e,), jnp.int32)]).reshape(NW, NCHUNKS, CHUNK)
    dst_p = jnp.concatenate([dst, pad_dst]).reshape(NW, NCHUNKS, CHUNK)
    zeros_tab = jnp.zeros((NPAD, HIDDEN), jnp.float32)

    y0 = pl.pallas_call(
        _proj_body,
        out_shape=jax.ShapeDtypeStruct((N_NODES, HIDDEN), jnp.float32),
    )(x, W1a)

    agg1 = _segsum(y0, zeros_tab, src_p, dst_p)

    z0 = pl.pallas_call(
        _mid_body,
        out_shape=jax.ShapeDtypeStruct((N_NODES, HIDDEN), jnp.float32),
    )(y0, agg1, b1a.reshape(1, HIDDEN), W1b, b1b.reshape(1, HIDDEN),
      gamma.reshape(1, HIDDEN), beta.reshape(1, HIDDEN), W2)

    agg2 = _segsum(z0, zeros_tab, src_p, dst_p)

    out = pl.pallas_call(
        _post_body,
        out_shape=jax.ShapeDtypeStruct((N_NODES, HIDDEN), jnp.float32),
    )(z0, agg2, b2.reshape(1, HIDDEN))
    return out


# speedup vs baseline: 17.2580x; 1.1373x over previous
import functools

import jax
import jax.numpy as jnp
from jax import lax
from jax.experimental import pallas as pl
from jax.experimental.pallas import tpu as pltpu
from jax.experimental.pallas import tpu_sc as plsc

N_NODES = 10000
D_FEAT = 128
HIDDEN = 16

NC = 2
NS = 16
CHUNK = 100
TOTAL_CHUNKS = 3200
C0 = 124
C1 = 76
SPLIT = NS * C0
ROWS_PER_TILE = N_NODES // NS
NBUF = 4
PREFETCH = 2

_MESH = plsc.VectorSubcoreMesh(
    core_axis_name="c", subcore_axis_name="s", num_cores=NC, num_subcores=NS
)


def _segsum_body(y0_hbm, zeros_hbm, edges_hbm, out_hbm,
                 sidx, didx, rows, acc, gsems, ssems):
    c = lax.axis_index("c")
    s = lax.axis_index("s")
    r0 = s * ROWS_PER_TILE

    pltpu.sync_copy(zeros_hbm.at[pl.ds(r0, ROWS_PER_TILE)],
                    acc.at[pl.ds(r0, ROWS_PER_TILE)])

    @pl.when(c == 0)
    def _():
        pltpu.sync_copy(edges_hbm.at[0, pl.ds(s * C0, C0)], sidx)
        pltpu.sync_copy(edges_hbm.at[1, pl.ds(s * C0, C0)], didx)

    @pl.when(c == 1)
    def _():
        pltpu.sync_copy(edges_hbm.at[0, pl.ds(SPLIT + s * C1, C1)],
                        sidx.at[pl.ds(0, C1)])
        pltpu.sync_copy(edges_hbm.at[1, pl.ds(SPLIT + s * C1, C1)],
                        didx.at[pl.ds(0, C1)])

    nch = jnp.where(c == 0, C0, C1)
    plsc.subcore_barrier()

    for b in range(PREFETCH):
        pltpu.async_copy(y0_hbm.at[sidx.at[b]], rows[b], gsems[b])

    def step(i, carry):
        for b in range(NBUF):
            j = i * NBUF + b
            pltpu.make_async_copy(y0_hbm.at[sidx.at[j]], rows[b],
                                  gsems[b]).wait()
            pltpu.async_copy(rows[b], acc.at[didx.at[j]], ssems[b], add=True)

            bn = (b + PREFETCH) % NBUF

            @pl.when(j >= NBUF - PREFETCH)
            def _():
                pltpu.make_async_copy(rows[bn], acc.at[didx.at[0]],
                                      ssems[bn]).wait()

            @pl.when(j + PREFETCH < nch)
            def _():
                pltpu.async_copy(y0_hbm.at[sidx.at[j + PREFETCH]], rows[bn],
                                 gsems[bn])
        return carry

    lax.fori_loop(0, nch // NBUF, step, 0)

    for b in range(PREFETCH, NBUF):
        pltpu.make_async_copy(rows[b], acc.at[didx.at[0]], ssems[b]).wait()

    plsc.subcore_barrier()
    pltpu.sync_copy(acc.at[pl.ds(r0, ROWS_PER_TILE)],
                    out_hbm.at[c, pl.ds(r0, ROWS_PER_TILE)])


_segsum = pl.kernel(
    _segsum_body,
    out_type=jax.ShapeDtypeStruct((NC, N_NODES, HIDDEN), jnp.float32),
    mesh=_MESH,
    scratch_types=[
        pltpu.VMEM((C0, CHUNK), jnp.int32),
        pltpu.VMEM((C0, CHUNK), jnp.int32),
        tuple(pltpu.VMEM((CHUNK, HIDDEN), jnp.float32)
              for _ in range(NBUF)),
        pltpu.VMEM_SHARED((N_NODES, HIDDEN), jnp.float32),
        tuple(pltpu.SemaphoreType.DMA for _ in range(NBUF)),
        tuple(pltpu.SemaphoreType.DMA for _ in range(NBUF)),
    ],
    compiler_params=pltpu.CompilerParams(use_tc_tiling_on_sc=False),
)


def _proj_body(x_ref, w_ref, o_ref):
    o_ref[...] = jnp.dot(x_ref[...], w_ref[...],
                         preferred_element_type=jnp.float32)


def _mid_body(y0_ref, agg_ref, b1a_ref, w1b_ref, b1b_ref, gamma_ref,
              beta_ref, w2_ref, o_ref):
    pre = y0_ref[...] + agg_ref[0] + agg_ref[1] + b1a_ref[...]
    a = jnp.maximum(pre, 0.0)
    h = jnp.dot(a, w1b_ref[...], preferred_element_type=jnp.float32) \
        + b1b_ref[...]
    mean = jnp.mean(h, axis=0, keepdims=True)
    var = jnp.mean((h - mean) ** 2, axis=0, keepdims=True)
    hn = (h - mean) * lax.rsqrt(var + 1e-5) * gamma_ref[...] + beta_ref[...]
    hn = jnp.maximum(hn, 0.0)
    o_ref[...] = jnp.dot(hn, w2_ref[...], preferred_element_type=jnp.float32)


def _post_body(z0_ref, agg_ref, b2_ref, o_ref):
    t = z0_ref[...] + agg_ref[0] + agg_ref[1] + b2_ref[...]
    m = jnp.max(t, axis=1, keepdims=True)
    lse = jnp.log(jnp.sum(jnp.exp(t - m), axis=1, keepdims=True))
    o_ref[...] = t - m - lse


def kernel(x, edge_index, W1a, b1a, W1b, b1b, gamma, beta, W2, b2):
    edges3 = edge_index.astype(jnp.int32).reshape(2, TOTAL_CHUNKS, CHUNK)
    zeros_tab = jnp.zeros((N_NODES, HIDDEN), jnp.float32)

    y0 = pl.pallas_call(
        _proj_body,
        out_shape=jax.ShapeDtypeStruct((N_NODES, HIDDEN), jnp.float32),
    )(x, W1a)

    agg1 = _segsum(y0, zeros_tab, edges3)

    z0 = pl.pallas_call(
        _mid_body,
        out_shape=jax.ShapeDtypeStruct((N_NODES, HIDDEN), jnp.float32),
    )(y0, agg1, b1a.reshape(1, HIDDEN), W1b, b1b.reshape(1, HIDDEN),
      gamma.reshape(1, HIDDEN), beta.reshape(1, HIDDEN), W2)

    agg2 = _segsum(z0, zeros_tab, edges3)

    out = pl.pallas_call(
        _post_body,
        out_shape=jax.ShapeDtypeStruct((N_NODES, HIDDEN), jnp.float32),
    )(z0, agg2, b2.reshape(1, HIDDEN))
    return out


# speedup vs baseline: 22.4292x; 1.2996x over previous
import functools

import jax
import jax.numpy as jnp
from jax import lax
from jax.experimental import pallas as pl
from jax.experimental.pallas import tpu as pltpu
from jax.experimental.pallas import tpu_sc as plsc

N_NODES = 10000
D_FEAT = 128
HIDDEN = 16
FOLD = 8
NPAD = 10240
NF = NPAD // FOLD
NF_REAL = N_NODES // FOLD
N_TRASH = NPAD - N_NODES

NC = 2
NS = 16
CHUNK = 100
TOTAL_CHUNKS = 3200
C0 = 100
C1 = 100
SPLIT = NS * C0
ROWS_PER_TILE = NPAD // NS
NBUF = 4
PREFETCH = 2

_MESH = plsc.VectorSubcoreMesh(
    core_axis_name="c", subcore_axis_name="s", num_cores=NC, num_subcores=NS
)


def _segsum_body(tab_hbm, zeros_hbm, edges_hbm, out_hbm,
                 sidx, didx, rows, acc, gsems, ssems):
    c = lax.axis_index("c")
    s = lax.axis_index("s")
    r0 = s * ROWS_PER_TILE

    @pl.when(c == 0)
    def _():
        pltpu.sync_copy(tab_hbm.at[pl.ds(r0, ROWS_PER_TILE)],
                        acc.at[pl.ds(r0, ROWS_PER_TILE)])
        pltpu.sync_copy(edges_hbm.at[0, pl.ds(s * C0, C0)], sidx)
        pltpu.sync_copy(edges_hbm.at[1, pl.ds(s * C0, C0)], didx)

    @pl.when(c == 1)
    def _():
        pltpu.sync_copy(zeros_hbm.at[pl.ds(r0, ROWS_PER_TILE)],
                        acc.at[pl.ds(r0, ROWS_PER_TILE)])
        pltpu.sync_copy(edges_hbm.at[0, pl.ds(SPLIT + s * C1, C1)],
                        sidx.at[pl.ds(0, C1)])
        pltpu.sync_copy(edges_hbm.at[1, pl.ds(SPLIT + s * C1, C1)],
                        didx.at[pl.ds(0, C1)])

    nch = jnp.where(c == 0, C0, C1)
    plsc.subcore_barrier()

    for b in range(PREFETCH):
        pltpu.async_copy(tab_hbm.at[sidx.at[b]], rows[b], gsems[b])

    def step(i, carry):
        for b in range(NBUF):
            j = i * NBUF + b
            pltpu.make_async_copy(tab_hbm.at[sidx.at[j]], rows[b],
                                  gsems[b]).wait()
            pltpu.async_copy(rows[b], acc.at[didx.at[j]], ssems[b], add=True)

            bn = (b + PREFETCH) % NBUF

            @pl.when(j >= NBUF - PREFETCH)
            def _():
                pltpu.make_async_copy(rows[bn], acc.at[didx.at[0]],
                                      ssems[bn]).wait()

            @pl.when(j + PREFETCH < nch)
            def _():
                pltpu.async_copy(tab_hbm.at[sidx.at[j + PREFETCH]], rows[bn],
                                 gsems[bn])
        return carry

    lax.fori_loop(0, nch // NBUF, step, 0)

    for b in range(PREFETCH, NBUF):
        pltpu.make_async_copy(rows[b], acc.at[didx.at[0]], ssems[b]).wait()

    plsc.subcore_barrier()
    pltpu.sync_copy(acc.at[pl.ds(r0, ROWS_PER_TILE)],
                    out_hbm.at[c, pl.ds(r0, ROWS_PER_TILE)])


_segsum = pl.kernel(
    _segsum_body,
    out_type=jax.ShapeDtypeStruct((NC, NPAD, HIDDEN), jnp.float32),
    mesh=_MESH,
    scratch_types=[
        pltpu.VMEM((max(C0, C1), CHUNK), jnp.int32),
        pltpu.VMEM((max(C0, C1), CHUNK), jnp.int32),
        tuple(pltpu.VMEM((CHUNK, HIDDEN), jnp.float32)
              for _ in range(NBUF)),
        pltpu.VMEM_SHARED((NPAD, HIDDEN), jnp.float32),
        tuple(pltpu.SemaphoreType.DMA for _ in range(NBUF)),
        tuple(pltpu.SemaphoreType.DMA for _ in range(NBUF)),
    ],
    compiler_params=pltpu.CompilerParams(use_tc_tiling_on_sc=False),
)


def _fold16(v128):
    out = v128[0:16]
    for i in range(1, FOLD):
        out = out + v128[16 * i:16 * (i + 1)]
    return out


def _proj_body(x_ref, w_ref, o_ref):
    o_ref[...] = jnp.dot(x_ref[...], w_ref[...],
                         preferred_element_type=jnp.float32)


def _mid_body(agg_ref, b1a_ref, w1bbd_ref, b1b_ref, gamma_ref,
              beta_ref, w2bd_ref, o_ref):
    pre = agg_ref[0] + agg_ref[1] + b1a_ref[...]
    a = jnp.maximum(pre, 0.0)
    h = jnp.dot(a, w1bbd_ref[...], preferred_element_type=jnp.float32) \
        + b1b_ref[...]
    v = (jnp.dot(jnp.maximum(b1a_ref[...], 0.0), w1bbd_ref[...],
                 preferred_element_type=jnp.float32) + b1b_ref[...])[0]
    s16 = _fold16(jnp.sum(h, axis=0))
    q16 = _fold16(jnp.sum(h * h, axis=0))
    v16 = v[0:16]
    mean16 = (s16 - N_TRASH * v16) / N_NODES
    var16 = (q16 - N_TRASH * v16 * v16) / N_NODES - mean16 * mean16
    mean = jnp.concatenate([mean16] * FOLD)
    rstd = jnp.concatenate([lax.rsqrt(var16 + 1e-5)] * FOLD)
    hn = (h - mean) * rstd * gamma_ref[...] + beta_ref[...]
    hn = jnp.maximum(hn, 0.0)
    o_ref[...] = jnp.dot(hn, w2bd_ref[...],
                         preferred_element_type=jnp.float32)


def _post_body(agg_ref, b2_ref, o_ref):
    tf = agg_ref[0] + agg_ref[1] + b2_ref[...]
    outs = []
    for i in range(FOLD):
        t = tf[:, 16 * i:16 * (i + 1)]
        m = jnp.max(t, axis=1, keepdims=True)
        lse = jnp.log(jnp.sum(jnp.exp(t - m), axis=1, keepdims=True))
        outs.append(t - m - lse)
    o_ref[...] = jnp.concatenate(outs, axis=1)


def kernel(x, edge_index, W1a, b1a, W1b, b1b, gamma, beta, W2, b2):
    edges3 = edge_index.astype(jnp.int32).reshape(2, TOTAL_CHUNKS, CHUNK)
    zeros_tab = jnp.zeros((NPAD, HIDDEN), jnp.float32)
    eye8 = jnp.eye(FOLD, dtype=jnp.float32)
    w1b_bd = jnp.kron(eye8, W1b)
    w2_bd = jnp.kron(eye8, W2)
    b1a8 = jnp.tile(b1a, FOLD).reshape(1, FOLD * HIDDEN)
    b1b8 = jnp.tile(b1b, FOLD).reshape(1, FOLD * HIDDEN)
    gamma8 = jnp.tile(gamma, FOLD).reshape(1, FOLD * HIDDEN)
    beta8 = jnp.tile(beta, FOLD).reshape(1, FOLD * HIDDEN)
    b28 = jnp.tile(b2, FOLD).reshape(1, FOLD * HIDDEN)

    y0 = pl.pallas_call(
        _proj_body,
        out_shape=jax.ShapeDtypeStruct((N_NODES, HIDDEN), jnp.float32),
    )(x, W1a)
    y0t = jnp.concatenate(
        [y0, jnp.zeros((NPAD - N_NODES, HIDDEN), jnp.float32)])

    agg1 = _segsum(y0t, zeros_tab, edges3)

    z0f = pl.pallas_call(
        _mid_body,
        out_shape=jax.ShapeDtypeStruct((NF, FOLD * HIDDEN), jnp.float32),
    )(agg1.reshape(2, NF, FOLD * HIDDEN), b1a8, w1b_bd, b1b8, gamma8,
      beta8, w2_bd)

    agg2 = _segsum(z0f.reshape(NPAD, HIDDEN), zeros_tab, edges3)

    outf = pl.pallas_call(
        _post_body,
        out_shape=jax.ShapeDtypeStruct((NF, FOLD * HIDDEN), jnp.float32),
    )(agg2.reshape(2, NF, FOLD * HIDDEN), b28)
    return outf.reshape(NPAD, HIDDEN)[:N_NODES]


# speedup vs baseline: 26.1975x; 1.1680x over previous
import functools

import jax
import jax.numpy as jnp
from jax import lax
from jax.experimental import pallas as pl
from jax.experimental.pallas import tpu as pltpu
from jax.experimental.pallas import tpu_sc as plsc

N_NODES = 10000
D_FEAT = 128
HIDDEN = 16
FOLD = 8
NPAD = 10240
NF = NPAD // FOLD
NF_REAL = N_NODES // FOLD
N_TRASH = NPAD - N_NODES

NC = 2
NS = 16
CHUNK = 128
TOTAL_CHUNKS = 2500
NBUF = 4
PREFETCH = 2
MAXG = 20
BIG_TILES = 17
MAXCH = MAXG * NBUF
ROWS_PER_TILE = NPAD // NS

_MESH = plsc.VectorSubcoreMesh(
    core_axis_name="c", subcore_axis_name="s", num_cores=NC, num_subcores=NS
)


def _segsum_body(tab_hbm, zeros_hbm, edges_hbm, out_hbm,
                 sidx, didx, rows, acc, gsems, ssems):
    c = lax.axis_index("c")
    s = lax.axis_index("s")
    wid = c * NS + s
    r0 = s * ROWS_PER_TILE

    @pl.when(c == 0)
    def _():
        pltpu.sync_copy(tab_hbm.at[pl.ds(r0, ROWS_PER_TILE)],
                        acc.at[pl.ds(r0, ROWS_PER_TILE)])

    @pl.when(c == 1)
    def _():
        pltpu.sync_copy(zeros_hbm.at[pl.ds(r0, ROWS_PER_TILE)],
                        acc.at[pl.ds(r0, ROWS_PER_TILE)])

    base = NBUF * ((MAXG - 1) * wid + jnp.minimum(wid, BIG_TILES))
    nch = NBUF * ((MAXG - 1) + jnp.where(wid < BIG_TILES, 1, 0))

    @pl.when(wid < BIG_TILES)
    def _():
        pltpu.sync_copy(edges_hbm.at[0, pl.ds(base, MAXCH)], sidx)
        pltpu.sync_copy(edges_hbm.at[1, pl.ds(base, MAXCH)], didx)

    @pl.when(wid >= BIG_TILES)
    def _():
        pltpu.sync_copy(edges_hbm.at[0, pl.ds(base, MAXCH - NBUF)],
                        sidx.at[pl.ds(0, MAXCH - NBUF)])
        pltpu.sync_copy(edges_hbm.at[1, pl.ds(base, MAXCH - NBUF)],
                        didx.at[pl.ds(0, MAXCH - NBUF)])

    plsc.subcore_barrier()

    for b in range(PREFETCH):
        pltpu.async_copy(tab_hbm.at[sidx.at[b]], rows[b], gsems[b])

    def step(i, carry):
        for b in range(NBUF):
            j = i * NBUF + b
            pltpu.make_async_copy(tab_hbm.at[sidx.at[j]], rows[b],
                                  gsems[b]).wait()
            pltpu.async_copy(rows[b], acc.at[didx.at[j]], ssems[b], add=True)

            bn = (b + PREFETCH) % NBUF

            @pl.when(j >= NBUF - PREFETCH)
            def _():
                pltpu.make_async_copy(rows[bn], acc.at[didx.at[0]],
                                      ssems[bn]).wait()

            @pl.when(j + PREFETCH < nch)
            def _():
                pltpu.async_copy(tab_hbm.at[sidx.at[j + PREFETCH]], rows[bn],
                                 gsems[bn])
        return carry

    lax.fori_loop(0, nch // NBUF, step, 0)

    for b in range(PREFETCH, NBUF):
        pltpu.make_async_copy(rows[b], acc.at[didx.at[0]], ssems[b]).wait()

    plsc.subcore_barrier()
    pltpu.sync_copy(acc.at[pl.ds(r0, ROWS_PER_TILE)],
                    out_hbm.at[c, pl.ds(r0, ROWS_PER_TILE)])


_segsum = pl.kernel(
    _segsum_body,
    out_type=jax.ShapeDtypeStruct((NC, NPAD, HIDDEN), jnp.float32),
    mesh=_MESH,
    scratch_types=[
        pltpu.VMEM((MAXCH, CHUNK), jnp.int32),
        pltpu.VMEM((MAXCH, CHUNK), jnp.int32),
        tuple(pltpu.VMEM((CHUNK, HIDDEN), jnp.float32)
              for _ in range(NBUF)),
        pltpu.VMEM_SHARED((NPAD, HIDDEN), jnp.float32),
        tuple(pltpu.SemaphoreType.DMA for _ in range(NBUF)),
        tuple(pltpu.SemaphoreType.DMA for _ in range(NBUF)),
    ],
    compiler_params=pltpu.CompilerParams(use_tc_tiling_on_sc=False),
)


def _fold16(v128):
    out = v128[0:16]
    for i in range(1, FOLD):
        out = out + v128[16 * i:16 * (i + 1)]
    return out


def _proj_body(x_ref, w_ref, o_ref):
    o_ref[...] = jnp.dot(x_ref[...], w_ref[...],
                         preferred_element_type=jnp.float32)


def _mid_body(agg_ref, b1a_ref, w1bbd_ref, b1b_ref, gamma_ref,
              beta_ref, w2bd_ref, o_ref):
    pre = agg_ref[0] + agg_ref[1] + b1a_ref[...]
    a = jnp.maximum(pre, 0.0)
    h = jnp.dot(a, w1bbd_ref[...], preferred_element_type=jnp.float32) \
        + b1b_ref[...]
    v = (jnp.dot(jnp.maximum(b1a_ref[...], 0.0), w1bbd_ref[...],
                 preferred_element_type=jnp.float32) + b1b_ref[...])[0]
    s16 = _fold16(jnp.sum(h, axis=0))
    q16 = _fold16(jnp.sum(h * h, axis=0))
    v16 = v[0:16]
    mean16 = (s16 - N_TRASH * v16) / N_NODES
    var16 = (q16 - N_TRASH * v16 * v16) / N_NODES - mean16 * mean16
    mean = jnp.concatenate([mean16] * FOLD)
    rstd = jnp.concatenate([lax.rsqrt(var16 + 1e-5)] * FOLD)
    hn = (h - mean) * rstd * gamma_ref[...] + beta_ref[...]
    hn = jnp.maximum(hn, 0.0)
    o_ref[...] = jnp.dot(hn, w2bd_ref[...],
                         preferred_element_type=jnp.float32)


def _post_body(agg_ref, b2_ref, o_ref):
    tf = agg_ref[0] + agg_ref[1] + b2_ref[...]
    outs = []
    for i in range(FOLD):
        t = tf[:, 16 * i:16 * (i + 1)]
        m = jnp.max(t, axis=1, keepdims=True)
        lse = jnp.log(jnp.sum(jnp.exp(t - m), axis=1, keepdims=True))
        outs.append(t - m - lse)
    o_ref[...] = jnp.concatenate(outs, axis=1)


def kernel(x, edge_index, W1a, b1a, W1b, b1b, gamma, beta, W2, b2):
    edges3 = edge_index.astype(jnp.int32).reshape(2, TOTAL_CHUNKS, CHUNK)
    zeros_tab = jnp.zeros((NPAD, HIDDEN), jnp.float32)
    eye8 = jnp.eye(FOLD, dtype=jnp.float32)
    w1b_bd = jnp.kron(eye8, W1b)
    w2_bd = jnp.kron(eye8, W2)
    b1a8 = jnp.tile(b1a, FOLD).reshape(1, FOLD * HIDDEN)
    b1b8 = jnp.tile(b1b, FOLD).reshape(1, FOLD * HIDDEN)
    gamma8 = jnp.tile(gamma, FOLD).reshape(1, FOLD * HIDDEN)
    beta8 = jnp.tile(beta, FOLD).reshape(1, FOLD * HIDDEN)
    b28 = jnp.tile(b2, FOLD).reshape(1, FOLD * HIDDEN)

    y0 = pl.pallas_call(
        _proj_body,
        out_shape=jax.ShapeDtypeStruct((N_NODES, HIDDEN), jnp.float32),
    )(x, W1a)
    y0t = jnp.concatenate(
        [y0, jnp.zeros((NPAD - N_NODES, HIDDEN), jnp.float32)])

    agg1 = _segsum(y0t, zeros_tab, edges3)

    z0f = pl.pallas_call(
        _mid_body,
        out_shape=jax.ShapeDtypeStruct((NF, FOLD * HIDDEN), jnp.float32),
    )(agg1.reshape(2, NF, FOLD * HIDDEN), b1a8, w1b_bd, b1b8, gamma8,
      beta8, w2_bd)

    agg2 = _segsum(z0f.reshape(NPAD, HIDDEN), zeros_tab, edges3)

    outf = pl.pallas_call(
        _post_body,
        out_shape=jax.ShapeDtypeStruct((NF, FOLD * HIDDEN), jnp.float32),
    )(agg2.reshape(2, NF, FOLD * HIDDEN), b28)
    return outf.reshape(NPAD, HIDDEN)[:N_NODES]


# speedup vs baseline: 31.1211x; 1.1879x over previous
import functools

import jax
import jax.numpy as jnp
from jax import lax
from jax.experimental import pallas as pl
from jax.experimental.pallas import tpu as pltpu
from jax.experimental.pallas import tpu_sc as plsc

N_NODES = 10000
D_FEAT = 128
HIDDEN = 16
FOLD = 8
NPAD = 10240
NF = NPAD // FOLD
NF_REAL = N_NODES // FOLD
N_TRASH = NPAD - N_NODES

NC = 2
NS = 16
CHUNK = 128
TOTAL_CHUNKS = 2500
NBUF = 4
PREFETCH = 3
MAXG = 20
BIG_TILES = 17
MAXCH = MAXG * NBUF
ROWS_PER_TILE = NPAD // NS

_MESH = plsc.VectorSubcoreMesh(
    core_axis_name="c", subcore_axis_name="s", num_cores=NC, num_subcores=NS
)


def _segsum_body(tab_hbm, zeros_hbm, edges_hbm, out_hbm,
                 sidx, didx, rows, acc, gsems, ssems):
    c = lax.axis_index("c")
    s = lax.axis_index("s")
    wid = c * NS + s
    r0 = s * ROWS_PER_TILE

    @pl.when(c == 0)
    def _():
        pltpu.sync_copy(tab_hbm.at[pl.ds(r0, ROWS_PER_TILE)],
                        acc.at[pl.ds(r0, ROWS_PER_TILE)])

    @pl.when(c == 1)
    def _():
        pltpu.sync_copy(zeros_hbm.at[pl.ds(r0, ROWS_PER_TILE)],
                        acc.at[pl.ds(r0, ROWS_PER_TILE)])

    base = NBUF * ((MAXG - 1) * wid + jnp.minimum(wid, BIG_TILES))
    nch = NBUF * ((MAXG - 1) + jnp.where(wid < BIG_TILES, 1, 0))

    @pl.when(wid < BIG_TILES)
    def _():
        pltpu.sync_copy(edges_hbm.at[0, pl.ds(base, MAXCH)], sidx)
        pltpu.sync_copy(edges_hbm.at[1, pl.ds(base, MAXCH)], didx)

    @pl.when(wid >= BIG_TILES)
    def _():
        pltpu.sync_copy(edges_hbm.at[0, pl.ds(base, MAXCH - NBUF)],
                        sidx.at[pl.ds(0, MAXCH - NBUF)])
        pltpu.sync_copy(edges_hbm.at[1, pl.ds(base, MAXCH - NBUF)],
                        didx.at[pl.ds(0, MAXCH - NBUF)])

    plsc.subcore_barrier()

    for b in range(PREFETCH):
        pltpu.async_copy(tab_hbm.at[sidx.at[b]], rows[b], gsems[b])

    def step(i, carry):
        for b in range(NBUF):
            j = i * NBUF + b
            pltpu.make_async_copy(tab_hbm.at[sidx.at[j]], rows[b],
                                  gsems[b]).wait()
            pltpu.async_copy(rows[b], acc.at[didx.at[j]], ssems[b], add=True)

            bn = (b + PREFETCH) % NBUF

            @pl.when(j >= NBUF - PREFETCH)
            def _():
                pltpu.make_async_copy(rows[bn], acc.at[didx.at[0]],
                                      ssems[bn]).wait()

            @pl.when(j + PREFETCH < nch)
            def _():
                pltpu.async_copy(tab_hbm.at[sidx.at[j + PREFETCH]], rows[bn],
                                 gsems[bn])
        return carry

    lax.fori_loop(0, nch // NBUF, step, 0)

    for b in range(PREFETCH, NBUF):
        pltpu.make_async_copy(rows[b], acc.at[didx.at[0]], ssems[b]).wait()

    plsc.subcore_barrier()
    pltpu.sync_copy(acc.at[pl.ds(r0, ROWS_PER_TILE)],
                    out_hbm.at[c, pl.ds(r0, ROWS_PER_TILE)])


_segsum = pl.kernel(
    _segsum_body,
    out_type=jax.ShapeDtypeStruct((NC, NPAD, HIDDEN), jnp.float32),
    mesh=_MESH,
    scratch_types=[
        pltpu.VMEM((MAXCH, CHUNK), jnp.int32),
        pltpu.VMEM((MAXCH, CHUNK), jnp.int32),
        tuple(pltpu.VMEM((CHUNK, HIDDEN), jnp.float32)
              for _ in range(NBUF)),
        pltpu.VMEM_SHARED((NPAD, HIDDEN), jnp.float32),
        tuple(pltpu.SemaphoreType.DMA for _ in range(NBUF)),
        tuple(pltpu.SemaphoreType.DMA for _ in range(NBUF)),
    ],
    compiler_params=pltpu.CompilerParams(use_tc_tiling_on_sc=False),
)


def _fold16(v128):
    out = v128[0:16]
    for i in range(1, FOLD):
        out = out + v128[16 * i:16 * (i + 1)]
    return out


def _proj_body(x_ref, w_ref, o_ref):
    o_ref[pl.ds(0, N_NODES), :] = jnp.dot(
        x_ref[...], w_ref[...], preferred_element_type=jnp.float32)
    o_ref[pl.ds(N_NODES, NPAD - N_NODES), :] = jnp.zeros(
        (NPAD - N_NODES, HIDDEN), jnp.float32)


def _mid_body(agg_ref, b1a_ref, w1bbd_ref, b1b_ref, gamma_ref,
              beta_ref, w2bd_ref, o_ref):
    pre = agg_ref[0] + agg_ref[1] + b1a_ref[...]
    a = jnp.maximum(pre, 0.0)
    h = jnp.dot(a, w1bbd_ref[...], preferred_element_type=jnp.float32) \
        + b1b_ref[...]
    v = (jnp.dot(jnp.maximum(b1a_ref[...], 0.0), w1bbd_ref[...],
                 preferred_element_type=jnp.float32) + b1b_ref[...])[0]
    s16 = _fold16(jnp.sum(h, axis=0))
    q16 = _fold16(jnp.sum(h * h, axis=0))
    v16 = v[0:16]
    mean16 = (s16 - N_TRASH * v16) / N_NODES
    var16 = (q16 - N_TRASH * v16 * v16) / N_NODES - mean16 * mean16
    mean = jnp.concatenate([mean16] * FOLD)
    rstd = jnp.concatenate([lax.rsqrt(var16 + 1e-5)] * FOLD)
    hn = (h - mean) * rstd * gamma_ref[...] + beta_ref[...]
    hn = jnp.maximum(hn, 0.0)
    o_ref[...] = jnp.dot(hn, w2bd_ref[...],
                         preferred_element_type=jnp.float32)


def _post_body(agg_ref, b2_ref, o_ref):
    tf = agg_ref[0] + agg_ref[1] + b2_ref[...]
    outs = []
    for i in range(FOLD):
        t = tf[:, 16 * i:16 * (i + 1)]
        m = jnp.max(t, axis=1, keepdims=True)
        lse = jnp.log(jnp.sum(jnp.exp(t - m), axis=1, keepdims=True))
        outs.append(t - m - lse)
    o_ref[...] = jnp.concatenate(outs, axis=1)


def kernel(x, edge_index, W1a, b1a, W1b, b1b, gamma, beta, W2, b2):
    edges3 = edge_index.astype(jnp.int32).reshape(2, TOTAL_CHUNKS, CHUNK)
    zeros_tab = jnp.zeros((NPAD, HIDDEN), jnp.float32)
    eye8 = jnp.eye(FOLD, dtype=jnp.float32)
    w1b_bd = jnp.kron(eye8, W1b)
    w2_bd = jnp.kron(eye8, W2)
    b1a8 = jnp.tile(b1a, FOLD).reshape(1, FOLD * HIDDEN)
    b1b8 = jnp.tile(b1b, FOLD).reshape(1, FOLD * HIDDEN)
    gamma8 = jnp.tile(gamma, FOLD).reshape(1, FOLD * HIDDEN)
    beta8 = jnp.tile(beta, FOLD).reshape(1, FOLD * HIDDEN)
    b28 = jnp.tile(b2, FOLD).reshape(1, FOLD * HIDDEN)

    y0t = pl.pallas_call(
        _proj_body,
        out_shape=jax.ShapeDtypeStruct((NPAD, HIDDEN), jnp.float32),
    )(x, W1a)

    agg1 = _segsum(y0t, zeros_tab, edges3)

    z0f = pl.pallas_call(
        _mid_body,
        out_shape=jax.ShapeDtypeStruct((NF, FOLD * HIDDEN), jnp.float32),
    )(agg1.reshape(2, NF, FOLD * HIDDEN), b1a8, w1b_bd, b1b8, gamma8,
      beta8, w2_bd)

    agg2 = _segsum(z0f.reshape(NPAD, HIDDEN), zeros_tab, edges3)

    outf = pl.pallas_call(
        _post_body,
        out_shape=jax.ShapeDtypeStruct((NF, FOLD * HIDDEN), jnp.float32),
    )(agg2.reshape(2, NF, FOLD * HIDDEN), b28)
    return outf.reshape(NPAD, HIDDEN)[:N_NODES]


# speedup vs baseline: 36.2175x; 1.1638x over previous
import functools

import jax
import jax.numpy as jnp
from jax import lax
from jax.experimental import pallas as pl
from jax.experimental.pallas import tpu as pltpu
from jax.experimental.pallas import tpu_sc as plsc

N_NODES = 10000
D_FEAT = 128
HIDDEN = 16
FOLD = 8
NPAD = 10240
NF = NPAD // FOLD
NF_REAL = N_NODES // FOLD
N_TRASH = NPAD - N_NODES

NC = 2
NS = 16
CHUNK = 128
TOTAL_CHUNKS = 2500
NBUF = 8
PREFETCH = 6
MAXG = 10
BIG_TILES = 24
MAXCH = MAXG * NBUF
LEFT0 = 312 * NBUF
NLEFT = TOTAL_CHUNKS - LEFT0
ROWS_PER_TILE = NPAD // NS

_MESH = plsc.VectorSubcoreMesh(
    core_axis_name="c", subcore_axis_name="s", num_cores=NC, num_subcores=NS
)


def _segsum_body(tab_hbm, zeros_hbm, edges_hbm, out_hbm,
                 sidx, didx, rows, acc, gsems, ssems):
    c = lax.axis_index("c")
    s = lax.axis_index("s")
    wid = c * NS + s
    r0 = s * ROWS_PER_TILE

    @pl.when(c == 0)
    def _():
        pltpu.sync_copy(tab_hbm.at[pl.ds(r0, ROWS_PER_TILE)],
                        acc.at[pl.ds(r0, ROWS_PER_TILE)])

    @pl.when(c == 1)
    def _():
        pltpu.sync_copy(zeros_hbm.at[pl.ds(r0, ROWS_PER_TILE)],
                        acc.at[pl.ds(r0, ROWS_PER_TILE)])

    base = NBUF * ((MAXG - 1) * wid + jnp.minimum(wid, BIG_TILES))
    nch = NBUF * ((MAXG - 1) + jnp.where(wid < BIG_TILES, 1, 0))

    @pl.when(wid < BIG_TILES)
    def _():
        pltpu.sync_copy(edges_hbm.at[0, pl.ds(base, MAXCH)], sidx)
        pltpu.sync_copy(edges_hbm.at[1, pl.ds(base, MAXCH)], didx)

    @pl.when(wid >= BIG_TILES)
    def _():
        pltpu.sync_copy(edges_hbm.at[0, pl.ds(base, MAXCH - NBUF)],
                        sidx.at[pl.ds(0, MAXCH - NBUF)])
        pltpu.sync_copy(edges_hbm.at[1, pl.ds(base, MAXCH - NBUF)],
                        didx.at[pl.ds(0, MAXCH - NBUF)])

    @pl.when(wid == NC * NS - 1)
    def _():
        pltpu.sync_copy(edges_hbm.at[0, pl.ds(LEFT0, NLEFT)],
                        sidx.at[pl.ds(MAXCH - NBUF, NLEFT)])
        pltpu.sync_copy(edges_hbm.at[1, pl.ds(LEFT0, NLEFT)],
                        didx.at[pl.ds(MAXCH - NBUF, NLEFT)])

    plsc.subcore_barrier()

    @pl.when(wid == NC * NS - 1)
    def _():
        for k in range(NLEFT):
            pltpu.async_copy(tab_hbm.at[sidx.at[MAXCH - NBUF + k]],
                             rows[0], gsems[0]).wait()
            pltpu.sync_copy(rows[0], acc.at[didx.at[MAXCH - NBUF + k]],
                            add=True)

    for b in range(PREFETCH):
        pltpu.async_copy(tab_hbm.at[sidx.at[b]], rows[b], gsems[b])

    def step(i, carry):
        for b in range(NBUF):
            j = i * NBUF + b
            pltpu.make_async_copy(tab_hbm.at[sidx.at[j]], rows[b],
                                  gsems[b]).wait()
            pltpu.async_copy(rows[b], acc.at[didx.at[j]], ssems[b], add=True)

            bn = (b + PREFETCH) % NBUF

            @pl.when(j >= NBUF - PREFETCH)
            def _():
                pltpu.make_async_copy(rows[bn], acc.at[didx.at[0]],
                                      ssems[bn]).wait()

            @pl.when(j + PREFETCH < nch)
            def _():
                pltpu.async_copy(tab_hbm.at[sidx.at[j + PREFETCH]], rows[bn],
                                 gsems[bn])
        return carry

    lax.fori_loop(0, nch // NBUF, step, 0)

    for b in range(PREFETCH, NBUF):
        pltpu.make_async_copy(rows[b], acc.at[didx.at[0]], ssems[b]).wait()

    plsc.subcore_barrier()
    pltpu.sync_copy(acc.at[pl.ds(r0, ROWS_PER_TILE)],
                    out_hbm.at[c, pl.ds(r0, ROWS_PER_TILE)])


_segsum = pl.kernel(
    _segsum_body,
    out_type=jax.ShapeDtypeStruct((NC, NPAD, HIDDEN), jnp.float32),
    mesh=_MESH,
    scratch_types=[
        pltpu.VMEM((MAXCH, CHUNK), jnp.int32),
        pltpu.VMEM((MAXCH, CHUNK), jnp.int32),
        tuple(pltpu.VMEM((CHUNK, HIDDEN), jnp.float32)
              for _ in range(NBUF)),
        pltpu.VMEM_SHARED((NPAD, HIDDEN), jnp.float32),
        tuple(pltpu.SemaphoreType.DMA for _ in range(NBUF)),
        tuple(pltpu.SemaphoreType.DMA for _ in range(NBUF)),
    ],
    compiler_params=pltpu.CompilerParams(use_tc_tiling_on_sc=False),
)


def _fold16(v128):
    out = v128[0:16]
    for i in range(1, FOLD):
        out = out + v128[16 * i:16 * (i + 1)]
    return out


def _proj_body(x_ref, w_ref, o_ref):
    o_ref[pl.ds(0, N_NODES), :] = jnp.dot(
        x_ref[...], w_ref[...], preferred_element_type=jnp.float32)
    o_ref[pl.ds(N_NODES, NPAD - N_NODES), :] = jnp.zeros(
        (NPAD - N_NODES, HIDDEN), jnp.float32)


def _mid_body(agg_ref, b1a_ref, w1bbd_ref, b1b_ref, gamma_ref,
              beta_ref, w2bd_ref, o_ref):
    pre = agg_ref[0] + agg_ref[1] + b1a_ref[...]
    a = jnp.maximum(pre, 0.0)
    h = jnp.dot(a, w1bbd_ref[...], preferred_element_type=jnp.float32) \
        + b1b_ref[...]
    v = (jnp.dot(jnp.maximum(b1a_ref[...], 0.0), w1bbd_ref[...],
                 preferred_element_type=jnp.float32) + b1b_ref[...])[0]
    s16 = _fold16(jnp.sum(h, axis=0))
    q16 = _fold16(jnp.sum(h * h, axis=0))
    v16 = v[0:16]
    mean16 = (s16 - N_TRASH * v16) / N_NODES
    var16 = (q16 - N_TRASH * v16 * v16) / N_NODES - mean16 * mean16
    mean = jnp.concatenate([mean16] * FOLD)
    rstd = jnp.concatenate([lax.rsqrt(var16 + 1e-5)] * FOLD)
    hn = (h - mean) * rstd * gamma_ref[...] + beta_ref[...]
    hn = jnp.maximum(hn, 0.0)
    o_ref[...] = jnp.dot(hn, w2bd_ref[...],
                         preferred_element_type=jnp.float32)


def _post_body(agg_ref, b2_ref, o_ref):
    tf = agg_ref[0] + agg_ref[1] + b2_ref[...]
    outs = []
    for i in range(FOLD):
        t = tf[:, 16 * i:16 * (i + 1)]
        m = jnp.max(t, axis=1, keepdims=True)
        lse = jnp.log(jnp.sum(jnp.exp(t - m), axis=1, keepdims=True))
        outs.append(t - m - lse)
    o_ref[...] = jnp.concatenate(outs, axis=1)


def kernel(x, edge_index, W1a, b1a, W1b, b1b, gamma, beta, W2, b2):
    edges3 = edge_index.astype(jnp.int32).reshape(2, TOTAL_CHUNKS, CHUNK)
    zeros_tab = jnp.zeros((NPAD, HIDDEN), jnp.float32)
    eye8 = jnp.eye(FOLD, dtype=jnp.float32)
    w1b_bd = jnp.kron(eye8, W1b)
    w2_bd = jnp.kron(eye8, W2)
    b1a8 = jnp.tile(b1a, FOLD).reshape(1, FOLD * HIDDEN)
    b1b8 = jnp.tile(b1b, FOLD).reshape(1, FOLD * HIDDEN)
    gamma8 = jnp.tile(gamma, FOLD).reshape(1, FOLD * HIDDEN)
    beta8 = jnp.tile(beta, FOLD).reshape(1, FOLD * HIDDEN)
    b28 = jnp.tile(b2, FOLD).reshape(1, FOLD * HIDDEN)

    y0t = pl.pallas_call(
        _proj_body,
        out_shape=jax.ShapeDtypeStruct((NPAD, HIDDEN), jnp.float32),
    )(x, W1a)

    agg1 = _segsum(y0t, zeros_tab, edges3)

    z0f = pl.pallas_call(
        _mid_body,
        out_shape=jax.ShapeDtypeStruct((NF, FOLD * HIDDEN), jnp.float32),
    )(agg1.reshape(2, NF, FOLD * HIDDEN), b1a8, w1b_bd, b1b8, gamma8,
      beta8, w2_bd)

    agg2 = _segsum(z0f.reshape(NPAD, HIDDEN), zeros_tab, edges3)

    outf = pl.pallas_call(
        _post_body,
        out_shape=jax.ShapeDtypeStruct((NF, FOLD * HIDDEN), jnp.float32),
    )(agg2.reshape(2, NF, FOLD * HIDDEN), b28)
    return outf.reshape(NPAD, HIDDEN)[:N_NODES]


# speedup vs baseline: 36.3101x; 1.0026x over previous
import functools

import jax
import jax.numpy as jnp
from jax import lax
from jax.experimental import pallas as pl
from jax.experimental.pallas import tpu as pltpu
from jax.experimental.pallas import tpu_sc as plsc

N_NODES = 10000
D_FEAT = 128
HIDDEN = 16
FOLD = 8
NPAD = 10240
NF = NPAD // FOLD
NF_REAL = N_NODES // FOLD
N_TRASH = NPAD - N_NODES

NC = 2
NS = 16
CHUNK = 128
TOTAL_CHUNKS = 2500
NBUF = 8
PREFETCH = 7
MAXG = 10
BIG_TILES = 24
MAXCH = MAXG * NBUF
LEFT0 = (BIG_TILES * MAXG + (NC * NS - BIG_TILES) * (MAXG - 1)) * NBUF
NLEFT = TOTAL_CHUNKS - LEFT0
ROWS_PER_TILE = NPAD // NS

_MESH = plsc.VectorSubcoreMesh(
    core_axis_name="c", subcore_axis_name="s", num_cores=NC, num_subcores=NS
)


def _segsum_body(tab_hbm, zeros_hbm, edges_hbm, out_hbm,
                 sidx, didx, rows, acc, gsems, ssems):
    c = lax.axis_index("c")
    s = lax.axis_index("s")
    wid = c * NS + s
    r0 = s * ROWS_PER_TILE

    @pl.when(c == 0)
    def _():
        pltpu.sync_copy(tab_hbm.at[pl.ds(r0, ROWS_PER_TILE)],
                        acc.at[pl.ds(r0, ROWS_PER_TILE)])

    @pl.when(c == 1)
    def _():
        pltpu.sync_copy(zeros_hbm.at[pl.ds(r0, ROWS_PER_TILE)],
                        acc.at[pl.ds(r0, ROWS_PER_TILE)])

    base = NBUF * ((MAXG - 1) * wid + jnp.minimum(wid, BIG_TILES))
    nch = NBUF * ((MAXG - 1) + jnp.where(wid < BIG_TILES, 1, 0))

    @pl.when(wid < BIG_TILES)
    def _():
        pltpu.sync_copy(edges_hbm.at[0, pl.ds(base, MAXCH)], sidx)
        pltpu.sync_copy(edges_hbm.at[1, pl.ds(base, MAXCH)], didx)

    @pl.when(wid >= BIG_TILES)
    def _():
        pltpu.sync_copy(edges_hbm.at[0, pl.ds(base, MAXCH - NBUF)],
                        sidx.at[pl.ds(0, MAXCH - NBUF)])
        pltpu.sync_copy(edges_hbm.at[1, pl.ds(base, MAXCH - NBUF)],
                        didx.at[pl.ds(0, MAXCH - NBUF)])

    @pl.when(wid == NC * NS - 1)
    def _():
        pltpu.sync_copy(edges_hbm.at[0, pl.ds(LEFT0, NLEFT)],
                        sidx.at[pl.ds(MAXCH - NBUF, NLEFT)])
        pltpu.sync_copy(edges_hbm.at[1, pl.ds(LEFT0, NLEFT)],
                        didx.at[pl.ds(MAXCH - NBUF, NLEFT)])

    plsc.subcore_barrier()

    @pl.when(wid == NC * NS - 1)
    def _():
        for k in range(NLEFT):
            pltpu.async_copy(tab_hbm.at[sidx.at[MAXCH - NBUF + k]],
                             rows[0], gsems[0]).wait()
            pltpu.sync_copy(rows[0], acc.at[didx.at[MAXCH - NBUF + k]],
                            add=True)

    for b in range(PREFETCH):
        pltpu.async_copy(tab_hbm.at[sidx.at[b]], rows[b], gsems[b])

    def step(i, carry):
        for b in range(NBUF):
            j = i * NBUF + b
            pltpu.make_async_copy(tab_hbm.at[sidx.at[j]], rows[b],
                                  gsems[b]).wait()
            pltpu.async_copy(rows[b], acc.at[didx.at[j]], ssems[b], add=True)

            bn = (b + PREFETCH) % NBUF

            @pl.when(j >= NBUF - PREFETCH)
            def _():
                pltpu.make_async_copy(rows[bn], acc.at[didx.at[0]],
                                      ssems[bn]).wait()

            @pl.when(j + PREFETCH < nch)
            def _():
                pltpu.async_copy(tab_hbm.at[sidx.at[j + PREFETCH]], rows[bn],
                                 gsems[bn])
        return carry

    lax.fori_loop(0, nch // NBUF, step, 0)

    for b in range(PREFETCH, NBUF):
        pltpu.make_async_copy(rows[b], acc.at[didx.at[0]], ssems[b]).wait()

    plsc.subcore_barrier()
    pltpu.sync_copy(acc.at[pl.ds(r0, ROWS_PER_TILE)],
                    out_hbm.at[c, pl.ds(r0, ROWS_PER_TILE)])


_segsum = pl.kernel(
    _segsum_body,
    out_type=jax.ShapeDtypeStruct((NC, NPAD, HIDDEN), jnp.float32),
    mesh=_MESH,
    scratch_types=[
        pltpu.VMEM((MAXCH, CHUNK), jnp.int32),
        pltpu.VMEM((MAXCH, CHUNK), jnp.int32),
        tuple(pltpu.VMEM((CHUNK, HIDDEN), jnp.float32)
              for _ in range(NBUF)),
        pltpu.VMEM_SHARED((NPAD, HIDDEN), jnp.float32),
        tuple(pltpu.SemaphoreType.DMA for _ in range(NBUF)),
        tuple(pltpu.SemaphoreType.DMA for _ in range(NBUF)),
    ],
    compiler_params=pltpu.CompilerParams(use_tc_tiling_on_sc=False),
)


def _fold16(v128):
    out = v128[0:16]
    for i in range(1, FOLD):
        out = out + v128[16 * i:16 * (i + 1)]
    return out


def _proj_body(x_ref, w_ref, o_ref):
    o_ref[pl.ds(0, N_NODES), :] = jnp.dot(
        x_ref[...], w_ref[...], preferred_element_type=jnp.float32)
    o_ref[pl.ds(N_NODES, NPAD - N_NODES), :] = jnp.zeros(
        (NPAD - N_NODES, HIDDEN), jnp.float32)


def _mid_body(agg_ref, b1a_ref, w1bbd_ref, b1b_ref, gamma_ref,
              beta_ref, w2bd_ref, o_ref):
    pre = agg_ref[0] + agg_ref[1] + b1a_ref[...]
    a = jnp.maximum(pre, 0.0)
    h = jnp.dot(a, w1bbd_ref[...], preferred_element_type=jnp.float32) \
        + b1b_ref[...]
    v = (jnp.dot(jnp.maximum(b1a_ref[...], 0.0), w1bbd_ref[...],
                 preferred_element_type=jnp.float32) + b1b_ref[...])[0]
    s16 = _fold16(jnp.sum(h, axis=0))
    q16 = _fold16(jnp.sum(h * h, axis=0))
    v16 = v[0:16]
    mean16 = (s16 - N_TRASH * v16) / N_NODES
    var16 = (q16 - N_TRASH * v16 * v16) / N_NODES - mean16 * mean16
    mean = jnp.concatenate([mean16] * FOLD)
    rstd = jnp.concatenate([lax.rsqrt(var16 + 1e-5)] * FOLD)
    hn = (h - mean) * rstd * gamma_ref[...] + beta_ref[...]
    hn = jnp.maximum(hn, 0.0)
    o_ref[...] = jnp.dot(hn, w2bd_ref[...],
                         preferred_element_type=jnp.float32)


def _post_body(agg_ref, b2_ref, o_ref):
    tf = agg_ref[0] + agg_ref[1] + b2_ref[...]
    outs = []
    for i in range(FOLD):
        t = tf[:, 16 * i:16 * (i + 1)]
        m = jnp.max(t, axis=1, keepdims=True)
        lse = jnp.log(jnp.sum(jnp.exp(t - m), axis=1, keepdims=True))
        outs.append(t - m - lse)
    o_ref[...] = jnp.concatenate(outs, axis=1)


def kernel(x, edge_index, W1a, b1a, W1b, b1b, gamma, beta, W2, b2):
    edges3 = edge_index.astype(jnp.int32).reshape(2, TOTAL_CHUNKS, CHUNK)
    zeros_tab = jnp.zeros((NPAD, HIDDEN), jnp.float32)
    eye8 = jnp.eye(FOLD, dtype=jnp.float32)
    w1b_bd = jnp.kron(eye8, W1b)
    w2_bd = jnp.kron(eye8, W2)
    b1a8 = jnp.tile(b1a, FOLD).reshape(1, FOLD * HIDDEN)
    b1b8 = jnp.tile(b1b, FOLD).reshape(1, FOLD * HIDDEN)
    gamma8 = jnp.tile(gamma, FOLD).reshape(1, FOLD * HIDDEN)
    beta8 = jnp.tile(beta, FOLD).reshape(1, FOLD * HIDDEN)
    b28 = jnp.tile(b2, FOLD).reshape(1, FOLD * HIDDEN)

    y0t = pl.pallas_call(
        _proj_body,
        out_shape=jax.ShapeDtypeStruct((NPAD, HIDDEN), jnp.float32),
    )(x, W1a)

    agg1 = _segsum(y0t, zeros_tab, edges3)

    z0f = pl.pallas_call(
        _mid_body,
        out_shape=jax.ShapeDtypeStruct((NF, FOLD * HIDDEN), jnp.float32),
    )(agg1.reshape(2, NF, FOLD * HIDDEN), b1a8, w1b_bd, b1b8, gamma8,
      beta8, w2_bd)

    agg2 = _segsum(z0f.reshape(NPAD, HIDDEN), zeros_tab, edges3)

    outf = pl.pallas_call(
        _post_body,
        out_shape=jax.ShapeDtypeStruct((NF, FOLD * HIDDEN), jnp.float32),
    )(agg2.reshape(2, NF, FOLD * HIDDEN), b28)
    return outf.reshape(NPAD, HIDDEN)[:N_NODES]
